# Initial kernel scaffold; baseline (speedup 1.0000x reference)
import jax
import jax.numpy as jnp
from jax.experimental import pallas as pl


def kernel(m_x_sim, m_x_sem, d_x_sim, ei_mm, ei_mmr, ei_dd, ei_ddr, ei_md, ei_dm, params):
    raise NotImplementedError("write your pallas kernel here")


# speedup vs baseline: 4.8837x; 4.8837x over previous
import jax
import jax.numpy as jnp
from jax import lax
from jax.experimental import pallas as pl
from jax.experimental.pallas import tpu as pltpu
from jax.experimental.pallas import tpu_sc as plsc

NM, ND, F = 10000, 2000, 128
NMP, NDP = 10240, 2048
CH = 128
NS = 16
f32 = jnp.float32
i32 = jnp.int32

EP_MM = 161792
EP_DD = 32768
EP_MD = 65536


def _pad_rows(x, n):
    return jnp.pad(x, ((0, n - x.shape[0]), (0, 0)))


def _pad_edges(ei, epad, dst_fill):
    src = ei[0].astype(i32)
    dst = ei[1].astype(i32)
    e = src.shape[0]
    return (jnp.pad(src, (0, epad - e)),
            jnp.pad(dst, (0, epad - e), constant_values=dst_fill))


def _make_counts(jobs, n_edges):
    mesh = plsc.VectorSubcoreMesh(core_axis_name="c", subcore_axis_name="s")
    nj = len(jobs)

    def body(*refs):
        d_refs = refs[:n_edges]
        ones_h = refs[n_edges]
        zeros_h = refs[n_edges + 1]
        o_refs = refs[n_edges + 2:n_edges + 2 + nj]
        acc1, ones_v, didx, vbuf, obuf = refs[n_edges + 2 + nj:]
        cid = lax.axis_index("c")
        sid = lax.axis_index("s")
        pltpu.sync_copy(ones_h, ones_v)
        pltpu.sync_copy(zeros_h, vbuf)
        for ji, job in enumerate(jobs):
            @pl.when(cid == job["core"])
            def _(job=job, oi=ji):
                rows = job["ndst"] // NS
                r0 = sid * rows
                plsc.subcore_barrier()
                pltpu.sync_copy(vbuf.at[pl.ds(0, rows)], acc1.at[pl.ds(r0, rows)])
                plsc.subcore_barrier()
                dref = d_refs[job["e"]]
                base0 = sid * job["chunks"] * CH

                def chunk(i, c):
                    b = base0 + i * CH
                    pltpu.sync_copy(dref.at[pl.ds(b, CH)], didx)
                    pltpu.sync_copy(ones_v, acc1.at[didx], add=True)
                    return c

                lax.fori_loop(0, job["chunks"], chunk, 0)
                plsc.subcore_barrier()
                pltpu.sync_copy(acc1.at[pl.ds(r0, rows)], obuf.at[pl.ds(0, rows)])
                pltpu.sync_copy(obuf.at[pl.ds(0, rows)], o_refs[oi].at[pl.ds(r0, rows)])

    out_type = [jax.ShapeDtypeStruct((job["ndst"],), f32) for job in jobs]
    scratch = [
        pltpu.VMEM_SHARED((NMP,), f32),
        pltpu.VMEM((CH,), f32),
        pltpu.VMEM((CH,), i32),
        pltpu.VMEM((NMP // NS,), f32),
        pltpu.VMEM((NMP // NS,), f32),
    ]
    return pl.kernel(body, out_type=out_type, mesh=mesh, scratch_types=scratch)


def _make_prop(jobs, n_tables, n_edges):
    mesh = plsc.VectorSubcoreMesh(core_axis_name="c", subcore_axis_name="s")
    nj = len(jobs)

    def body(*refs):
        t_refs = refs[:n_tables]
        s_refs = refs[n_tables:n_tables + n_edges]
        d_refs = refs[n_tables + n_edges:n_tables + 2 * n_edges]
        z_hbm = refs[n_tables + 2 * n_edges]
        o_refs = refs[n_tables + 2 * n_edges + 1:n_tables + 2 * n_edges + 1 + nj]
        acc, rowbuf, zbuf, sidx, didx, sem = refs[n_tables + 2 * n_edges + 1 + nj:]
        cid = lax.axis_index("c")
        sid = lax.axis_index("s")
        pltpu.sync_copy(z_hbm, zbuf)
        for ji, job in enumerate(jobs):
            @pl.when(cid == job["core"])
            def _(job=job, oi=ji):
                rows = job["ndst"] // NS
                r0 = sid * rows
                plsc.subcore_barrier()
                for z in range(rows // CH):
                    pltpu.sync_copy(zbuf, acc.at[pl.ds(r0 + z * CH, CH)])
                plsc.subcore_barrier()
                sref = s_refs[job["e"]]
                dref = d_refs[job["e"]]
                tref = t_refs[job["t"]]
                base0 = sid * job["chunks"] * CH

                def chunk(i, c):
                    b = base0 + i * CH
                    pltpu.sync_copy(sref.at[pl.ds(b, CH)], sidx)
                    pltpu.sync_copy(dref.at[pl.ds(b, CH)], didx)
                    pltpu.async_copy(tref.at[sidx], rowbuf, sem).wait()
                    pltpu.sync_copy(rowbuf, acc.at[didx], add=True)
                    return c

                lax.fori_loop(0, job["chunks"], chunk, 0)
                plsc.subcore_barrier()
                for z in range(rows // CH):
                    pltpu.sync_copy(acc.at[pl.ds(r0 + z * CH, CH)], rowbuf)
                    pltpu.sync_copy(rowbuf, o_refs[oi].at[pl.ds(r0 + z * CH, CH)])

    out_type = [jax.ShapeDtypeStruct((job["ndst"], F), f32) for job in jobs]
    scratch = [
        pltpu.VMEM_SHARED((NMP, F), f32),
        pltpu.VMEM((CH, F), f32),
        pltpu.VMEM((CH, F), f32),
        pltpu.VMEM((CH,), i32),
        pltpu.VMEM((CH,), i32),
        pltpu.SemaphoreType.DMA,
    ]
    return pl.kernel(body, out_type=out_type, mesh=mesh, scratch_types=scratch)


def _row(bm, w=F):
    return pl.BlockSpec((bm, w), lambda i: (i, 0))


def _vec(bm):
    return pl.BlockSpec((bm,), lambda i: (i,))


def _wmat():
    return pl.BlockSpec((F, F), lambda i: (0, 0))


def _bvec():
    return pl.BlockSpec((F,), lambda i: (0,))


def _dot(a, b):
    return jnp.dot(a, b, preferred_element_type=f32)


def _mlp_scale_kernel(x1, x2, c1, c2, w1a, b1a, w2a, b2a, w1b, b1b, w2b, b2b):
    n = x1.shape[0]
    bm = 2048
    grid = (n // bm,)

    def body(x1r, x2r, c1r, c2r, w1ar, b1ar, w2ar, b2ar, w1br, b1br, w2br, b2br,
             ya_o, yb_o, yad1_o, yad2_o, ybd1_o, ybd2_o):
        ya = _dot(jax.nn.relu(_dot(x1r[...], w1ar[...]) + b1ar[...]), w2ar[...]) + b2ar[...]
        yb = _dot(jax.nn.relu(_dot(x2r[...], w1br[...]) + b1br[...]), w2br[...]) + b2br[...]
        d1 = lax.rsqrt(c1r[...] + 1.0)[:, None]
        d2 = lax.rsqrt(c2r[...] + 1.0)[:, None]
        ya_o[...] = ya
        yb_o[...] = yb
        yad1_o[...] = ya * d1
        yad2_o[...] = ya * d2
        ybd1_o[...] = yb * d1
        ybd2_o[...] = yb * d2

    sds = jax.ShapeDtypeStruct((n, F), f32)
    return pl.pallas_call(
        body,
        grid=grid,
        in_specs=[_row(bm), _row(bm), _vec(bm), _vec(bm),
                  _wmat(), _bvec(), _wmat(), _bvec(),
                  _wmat(), _bvec(), _wmat(), _bvec()],
        out_specs=[_row(bm)] * 6,
        out_shape=[sds] * 6,
    )(x1, x2, c1, c2, w1a, b1a, w2a, b2a, w1b, b1b, w2b, b2b)


def _mlp_scale_kernel_single(x1, c1, c2, w1a, b1a, w2a, b2a):
    n = x1.shape[0]
    bm = 2048
    grid = (n // bm,)

    def body(x1r, c1r, c2r, w1ar, b1ar, w2ar, b2ar, y_o, yd1_o, yd2_o):
        y = _dot(jax.nn.relu(_dot(x1r[...], w1ar[...]) + b1ar[...]), w2ar[...]) + b2ar[...]
        d1 = lax.rsqrt(c1r[...] + 1.0)[:, None]
        d2 = lax.rsqrt(c2r[...] + 1.0)[:, None]
        y_o[...] = y
        yd1_o[...] = y * d1
        yd2_o[...] = y * d2

    sds = jax.ShapeDtypeStruct((n, F), f32)
    return pl.pallas_call(
        body,
        grid=grid,
        in_specs=[_row(bm), _vec(bm), _vec(bm), _wmat(), _bvec(), _wmat(), _bvec()],
        out_specs=[_row(bm)] * 3,
        out_shape=[sds] * 3,
    )(x1, c1, c2, w1a, b1a, w2a, b2a)


def _mid_kernel(agg_h0, agg_h1, agg_e0, agg_e1, agg_sage, xh, xe,
                c0, c1, csage, W0, b0, W1, b1, Wl, bl, Wr, with_sage_on_e=True):
    n = agg_h0.shape[0]
    bm = 2048
    grid = (n // bm,)

    def body(ah0, ah1, ae0, ae1, asg, xhr, xer, c0r, c1r, csr,
             W0r, b0r, W1r, b1r, Wlr, blr, Wrr,
             yh_o, ye_o, yhd0_o, yhd1_o, yed0_o, yed1_o):
        d0 = lax.rsqrt(c0r[...] + 1.0)[:, None]
        d1 = lax.rsqrt(c1r[...] + 1.0)[:, None]
        uh0 = d0 * ah0[...] + d0 * d0 * xhr[...]
        uh1 = d1 * ah1[...] + d1 * d1 * xhr[...]
        zh = _dot(uh0, W0r[...]) + b0r[...] + _dot(uh1, W1r[...]) + b1r[...]
        ue0 = d0 * ae0[...] + d0 * d0 * xer[...]
        ue1 = d1 * ae1[...] + d1 * d1 * xer[...]
        mean = asg[...] / jnp.maximum(csr[...], 1.0)[:, None]
        ze = (_dot(ue0, W0r[...]) + b0r[...] + _dot(ue1, W1r[...]) + b1r[...]
              + _dot(mean, Wlr[...]) + blr[...] + _dot(xer[...], Wrr[...]))
        yh = jax.nn.relu(zh)
        ye = jax.nn.relu(ze)
        yh_o[...] = yh
        ye_o[...] = ye
        yhd0_o[...] = yh * d0
        yhd1_o[...] = yh * d1
        yed0_o[...] = ye * d0
        yed1_o[...] = ye * d1

    sds = jax.ShapeDtypeStruct((n, F), f32)
    return pl.pallas_call(
        body,
        grid=grid,
        in_specs=[_row(bm)] * 7 + [_vec(bm)] * 3
        + [_wmat(), _bvec(), _wmat(), _bvec(), _wmat(), _bvec(), _wmat()],
        out_specs=[_row(bm)] * 6,
        out_shape=[sds] * 6,
    )(agg_h0, agg_h1, agg_e0, agg_e1, agg_sage, xh, xe, c0, c1, csage,
      W0, b0, W1, b1, Wl, bl, Wr)


def _post_kernel(agg_h0, agg_h1, agg_e0, agg_e1, agg_sage, xh, xe,
                 c0, c1, csage, W0, b0, W1, b1, Wl, bl, Wr,
                 fW1, fW2, fb):
    n = agg_h0.shape[0]
    bm = 2048
    grid = (n // bm,)

    def body(ah0, ah1, ae0, ae1, asg, xhr, xer, c0r, c1r, csr,
             W0r, b0r, W1r, b1r, Wlr, blr, Wrr, fW1r, fW2r, fbr, z_o):
        d0 = lax.rsqrt(c0r[...] + 1.0)[:, None]
        d1 = lax.rsqrt(c1r[...] + 1.0)[:, None]
        uh0 = d0 * ah0[...] + d0 * d0 * xhr[...]
        uh1 = d1 * ah1[...] + d1 * d1 * xhr[...]
        zs = _dot(uh0, W0r[...]) + b0r[...] + _dot(uh1, W1r[...]) + b1r[...]
        ue0 = d0 * ae0[...] + d0 * d0 * xer[...]
        ue1 = d1 * ae1[...] + d1 * d1 * xer[...]
        mean = asg[...] / jnp.maximum(csr[...], 1.0)[:, None]
        ze = (_dot(ue0, W0r[...]) + b0r[...] + _dot(ue1, W1r[...]) + b1r[...]
              + _dot(mean, Wlr[...]) + blr[...] + _dot(xer[...], Wrr[...]))
        z_o[...] = jax.nn.relu(_dot(zs, fW1r[...]) + _dot(ze, fW2r[...]) + fbr[...])

    return pl.pallas_call(
        body,
        grid=grid,
        in_specs=[_row(bm)] * 7 + [_vec(bm)] * 3
        + [_wmat(), _bvec(), _wmat(), _bvec(), _wmat(), _bvec(), _wmat(),
           _wmat(), _wmat(), _bvec()],
        out_specs=[_row(bm)],
        out_shape=[jax.ShapeDtypeStruct((n, F), f32)],
    )(agg_h0, agg_h1, agg_e0, agg_e1, agg_sage, xh, xe, c0, c1, csage,
      W0, b0, W1, b1, Wl, bl, Wr, fW1, fW2, fb)[0]


def _bilinear_kernel(zm, Wbil, zd):
    bm = 2048
    grid = (NMP // bm,)

    def body(zmr, Wr, zdr, o):
        t = _dot(zmr[...], Wr[...])
        o[...] = lax.dot_general(t, zdr[...], (((1,), (1,)), ((), ())),
                                 preferred_element_type=f32)

    return pl.pallas_call(
        body,
        grid=grid,
        in_specs=[_row(bm), _wmat(), pl.BlockSpec((NDP, F), lambda i: (0, 0))],
        out_specs=[pl.BlockSpec((bm, NDP), lambda i: (i, 0))],
        out_shape=[jax.ShapeDtypeStruct((NMP, NDP), f32)],
    )(zm, Wbil, zd)[0]


def kernel(m_x_sim, m_x_sem, d_x_sim, ei_mm, ei_mmr, ei_dd, ei_ddr, ei_md,
           ei_dm, params):
    p = params
    mxs = _pad_rows(m_x_sim, NMP)
    mxe = _pad_rows(m_x_sem, NMP)
    dxs = _pad_rows(d_x_sim, NDP)

    s_mm, d_mm = _pad_edges(ei_mm, EP_MM, NM)
    s_mmr, d_mmr = _pad_edges(ei_mmr, EP_MM, NM)
    s_dd, d_dd = _pad_edges(ei_dd, EP_DD, ND)
    s_ddr, d_ddr = _pad_edges(ei_ddr, EP_DD, ND)
    s_md, d_md = _pad_edges(ei_md, EP_MD, ND)
    s_dm, d_dm = _pad_edges(ei_dm, EP_MD, NM)

    ones_h = jnp.ones((CH,), f32)
    z1_h = jnp.zeros((NMP // NS,), f32)
    zrow_h = jnp.zeros((CH, F), f32)

    cnt_jobs = [
        dict(core=0, e=0, ndst=NMP, chunks=EP_MM // (NS * CH)),
        dict(core=0, e=2, ndst=NDP, chunks=EP_DD // (NS * CH)),
        dict(core=0, e=4, ndst=NDP, chunks=EP_MD // (NS * CH)),
        dict(core=1, e=1, ndst=NMP, chunks=EP_MM // (NS * CH)),
        dict(core=1, e=3, ndst=NDP, chunks=EP_DD // (NS * CH)),
        dict(core=1, e=5, ndst=NMP, chunks=EP_MD // (NS * CH)),
    ]
    cmm, cdd, cmd, cmmr, cddr, cdm = _make_counts(cnt_jobs, 6)(
        d_mm, d_mmr, d_dd, d_ddr, d_md, d_dm, ones_h, z1_h)

    ms, me, ms_mm, ms_mmr, me_mm, me_mmr = _mlp_scale_kernel(
        mxs, mxe, cmm, cmmr,
        p["msim_W1"], p["msim_b1"], p["msim_W2"], p["msim_b2"],
        p["msem_W1"], p["msem_b1"], p["msem_W2"], p["msem_b2"])
    ds, ds_dd, ds_ddr = _mlp_scale_kernel_single(
        dxs, cdd, cddr,
        p["dsim_W1"], p["dsim_b1"], p["dsim_W2"], p["dsim_b2"])

    ch_mm = EP_MM // (NS * CH)
    ch_dd = EP_DD // (NS * CH)
    ch_md = EP_MD // (NS * CH)
    l1_jobs = [
        dict(core=0, t=0, e=0, ndst=NMP, chunks=ch_mm),
        dict(core=0, t=1, e=1, ndst=NMP, chunks=ch_mm),
        dict(core=0, t=4, e=2, ndst=NDP, chunks=ch_dd),
        dict(core=0, t=6, e=4, ndst=NDP, chunks=ch_md),
        dict(core=1, t=2, e=0, ndst=NMP, chunks=ch_mm),
        dict(core=1, t=3, e=1, ndst=NMP, chunks=ch_mm),
        dict(core=1, t=5, e=3, ndst=NDP, chunks=ch_dd),
        dict(core=1, t=7, e=5, ndst=NMP, chunks=ch_md),
    ]
    (a1_mm_h, a1_mmr_h, a1_dd, a1_md,
     a1_mm_e, a1_mmr_e, a1_ddr, a1_dm) = _make_prop(l1_jobs, 8, 6)(
        ms_mm, ms_mmr, me_mm, me_mmr, ds_dd, ds_ddr, me, ds,
        s_mm, s_mmr, s_dd, s_ddr, s_md, s_dm,
        d_mm, d_mmr, d_dd, d_ddr, d_md, d_dm,
        zrow_h)

    m_h1, m_e1, mh_mm, mh_mmr, me1_mm, me1_mmr = _mid_kernel(
        a1_mm_h, a1_mmr_h, a1_mm_e, a1_mmr_e, a1_dm, ms, me,
        cmm, cmmr, cdm,
        p["g1_mm_W"], p["g1_mm_b"], p["g1_mmr_W"], p["g1_mmr_b"],
        p["g1_dm_Wl"], p["g1_dm_bl"], p["g1_dm_Wr"])
    d_h1, d_e1, dh_dd, dh_ddr, de_dd, de_ddr = _mid_kernel(
        a1_dd, a1_ddr, a1_dd, a1_ddr, a1_md, ds, ds,
        cdd, cddr, cmd,
        p["g1_dd_W"], p["g1_dd_b"], p["g1_ddr_W"], p["g1_ddr_b"],
        p["g1_md_Wl"], p["g1_md_bl"], p["g1_md_Wr"])

    l2_jobs = [
        dict(core=0, t=0, e=0, ndst=NMP, chunks=ch_mm),
        dict(core=0, t=1, e=1, ndst=NMP, chunks=ch_mm),
        dict(core=0, t=4, e=2, ndst=NDP, chunks=ch_dd),
        dict(core=0, t=5, e=3, ndst=NDP, chunks=ch_dd),
        dict(core=0, t=8, e=4, ndst=NDP, chunks=ch_md),
        dict(core=1, t=2, e=0, ndst=NMP, chunks=ch_mm),
        dict(core=1, t=3, e=1, ndst=NMP, chunks=ch_mm),
        dict(core=1, t=6, e=2, ndst=NDP, chunks=ch_dd),
        dict(core=1, t=7, e=3, ndst=NDP, chunks=ch_dd),
        dict(core=1, t=9, e=5, ndst=NMP, chunks=ch_md),
    ]
    (a2_mm_h, a2_mmr_h, a2_dd_h, a2_ddr_h, a2_md,
     a2_mm_e, a2_mmr_e, a2_dd_e, a2_ddr_e, a2_dm) = _make_prop(l2_jobs, 10, 6)(
        mh_mm, mh_mmr, me1_mm, me1_mmr, dh_dd, dh_ddr, de_dd, de_ddr,
        m_e1, d_e1,
        s_mm, s_mmr, s_dd, s_ddr, s_md, s_dm,
        d_mm, d_mmr, d_dd, d_ddr, d_md, d_dm,
        zrow_h)

    fm_W1, fm_W2 = p["fm_W"][:F], p["fm_W"][F:]
    fd_W1, fd_W2 = p["fd_W"][:F], p["fd_W"][F:]
    zd = _post_kernel(
        a2_dd_h, a2_ddr_h, a2_dd_e, a2_ddr_e, a2_md, d_h1, d_e1,
        cdd, cddr, cmd,
        p["g2_dd_W"], p["g2_dd_b"], p["g2_ddr_W"], p["g2_ddr_b"],
        p["g2_md_Wl"], p["g2_md_bl"], p["g2_md_Wr"],
        fd_W1, fd_W2, p["fd_b"])
    zm = _post_kernel(
        a2_mm_h, a2_mmr_h, a2_mm_e, a2_mmr_e, a2_dm, m_h1, m_e1,
        cmm, cmmr, cdm,
        p["g2_mm_W"], p["g2_mm_b"], p["g2_mmr_W"], p["g2_mmr_b"],
        p["g2_dm_Wl"], p["g2_dm_bl"], p["g2_dm_Wr"],
        fm_W1, fm_W2, p["fm_b"])
    logits = _bilinear_kernel(zm, p["Wbil"], zd)
    return logits[:NM, :ND]


# speedup vs baseline: 6.5857x; 1.3485x over previous
import jax
import jax.numpy as jnp
from jax import lax
from jax.experimental import pallas as pl
from jax.experimental.pallas import tpu as pltpu
from jax.experimental.pallas import tpu_sc as plsc

NM, ND, F = 10000, 2000, 128
NMP, NDP = 10240, 2048
CH = 128
NS = 16
SLAB = 40
f32 = jnp.float32
i32 = jnp.int32

EP_MM = 163840
EP_DD = 32768
EP_MD = 65536


def _pad_rows(x, n):
    return jnp.pad(x, ((0, n - x.shape[0]), (0, 0)))


def _pad_edges(ei, epad, dst_fill):
    src = ei[0].astype(i32)
    dst = ei[1].astype(i32)
    e = src.shape[0]
    return (jnp.pad(src, (0, epad - e)).reshape(epad // CH, CH),
            jnp.pad(dst, (0, epad - e),
                    constant_values=dst_fill).reshape(epad // CH, CH))


def _make_counts(jobs, n_edges, max_ch):
    mesh = plsc.VectorSubcoreMesh(core_axis_name="c", subcore_axis_name="s")
    nj = len(jobs)

    def body(*refs):
        d_refs = refs[:n_edges]
        ones_h = refs[n_edges]
        zeros_h = refs[n_edges + 1]
        o_refs = refs[n_edges + 2:n_edges + 2 + nj]
        acc1, ones_v, vbuf, obuf, didx2 = refs[n_edges + 2 + nj:]
        cid = lax.axis_index("c")
        sid = lax.axis_index("s")
        pltpu.sync_copy(ones_h, ones_v)
        pltpu.sync_copy(zeros_h, vbuf)
        for ji, job in enumerate(jobs):
            @pl.when(cid == job["core"])
            def _(job=job, oi=ji):
                rows = job["ndst"] // NS
                r0 = sid * rows
                nch = job["chunks"]
                plsc.subcore_barrier()
                pltpu.sync_copy(vbuf.at[pl.ds(0, rows)], acc1.at[pl.ds(r0, rows)])
                dref = d_refs[job["e"]]
                row0 = sid * nch
                pltpu.sync_copy(dref.at[pl.ds(row0, nch)], didx2.at[pl.ds(0, nch)])
                plsc.subcore_barrier()

                def chunk(i, c):
                    pltpu.sync_copy(ones_v, acc1.at[didx2.at[i]], add=True)
                    return c

                lax.fori_loop(0, nch, chunk, 0)
                plsc.subcore_barrier()
                pltpu.sync_copy(acc1.at[pl.ds(r0, rows)], obuf.at[pl.ds(0, rows)])
                pltpu.sync_copy(obuf.at[pl.ds(0, rows)], o_refs[oi].at[pl.ds(r0, rows)])

    out_type = [jax.ShapeDtypeStruct((job["ndst"],), f32) for job in jobs]
    scratch = [
        pltpu.VMEM_SHARED((NMP,), f32),
        pltpu.VMEM((CH,), f32),
        pltpu.VMEM((NMP // NS,), f32),
        pltpu.VMEM((NMP // NS,), f32),
        pltpu.VMEM((max_ch, CH), i32),
    ]
    return pl.kernel(body, out_type=out_type, mesh=mesh, scratch_types=scratch)


def _make_prop(jobs, n_tables, n_edges, max_ch):
    mesh = plsc.VectorSubcoreMesh(core_axis_name="c", subcore_axis_name="s")
    nj = len(jobs)

    def body(*refs):
        t_refs = refs[:n_tables]
        s_refs = refs[n_tables:n_tables + n_edges]
        d_refs = refs[n_tables + n_edges:n_tables + 2 * n_edges]
        z_hbm = refs[n_tables + 2 * n_edges]
        o_refs = refs[n_tables + 2 * n_edges + 1:n_tables + 2 * n_edges + 1 + nj]
        (acc, rb0, rb1, sidx2, didx2,
         sem0, sem1) = refs[n_tables + 2 * n_edges + 1 + nj:]
        cid = lax.axis_index("c")
        sid = lax.axis_index("s")
        slab = min(max_ch, SLAB)
        for ji, job in enumerate(jobs):
            @pl.when(cid == job["core"])
            def _(job=job, oi=ji):
                rows = job["ndst"] // NS
                r0 = sid * rows
                nch = job["chunks"]
                plsc.subcore_barrier()
                pltpu.sync_copy(z_hbm.at[pl.ds(0, rows)], acc.at[pl.ds(r0, rows)])
                sref = s_refs[job["e"]]
                dref = d_refs[job["e"]]
                tref = t_refs[job["t"]]
                row0 = sid * nch
                plsc.subcore_barrier()
                for s0 in range(0, nch, slab):
                    ns = min(slab, nch - s0)
                    pltpu.sync_copy(sref.at[pl.ds(row0 + s0, ns)],
                                    sidx2.at[pl.ds(0, ns)])
                    pltpu.sync_copy(dref.at[pl.ds(row0 + s0, ns)],
                                    didx2.at[pl.ds(0, ns)])
                    pltpu.async_copy(tref.at[sidx2.at[0]], rb0, sem0)

                    def chunk(i, c, ns=ns):
                        @pl.when(i % 2 == 0)
                        def _():
                            @pl.when(i + 1 < ns)
                            def _():
                                pltpu.async_copy(tref.at[sidx2.at[i + 1]], rb1, sem1)
                            pltpu.make_async_copy(tref.at[sidx2.at[i]], rb0, sem0).wait()
                            pltpu.sync_copy(rb0, acc.at[didx2.at[i]], add=True)

                        @pl.when(i % 2 == 1)
                        def _():
                            @pl.when(i + 1 < ns)
                            def _():
                                pltpu.async_copy(tref.at[sidx2.at[i + 1]], rb0, sem0)
                            pltpu.make_async_copy(tref.at[sidx2.at[i]], rb1, sem1).wait()
                            pltpu.sync_copy(rb1, acc.at[didx2.at[i]], add=True)
                        return c

                    lax.fori_loop(0, ns, chunk, 0)
                plsc.subcore_barrier()
                pltpu.sync_copy(acc.at[pl.ds(r0, rows)],
                                o_refs[oi].at[pl.ds(r0, rows)])

    out_type = [jax.ShapeDtypeStruct((job["ndst"], F), f32) for job in jobs]
    scratch = [
        pltpu.VMEM_SHARED((NMP, F), f32),
        pltpu.VMEM((CH, F), f32),
        pltpu.VMEM((CH, F), f32),
        pltpu.VMEM((min(max_ch, SLAB), CH), i32),
        pltpu.VMEM((min(max_ch, SLAB), CH), i32),
        pltpu.SemaphoreType.DMA,
        pltpu.SemaphoreType.DMA,
    ]
    return pl.kernel(body, out_type=out_type, mesh=mesh, scratch_types=scratch)


def _row(bm, w=F):
    return pl.BlockSpec((bm, w), lambda i: (i, 0))


def _vec(bm):
    return pl.BlockSpec((bm,), lambda i: (i,))


def _wmat():
    return pl.BlockSpec((F, F), lambda i: (0, 0))


def _bvec():
    return pl.BlockSpec((F,), lambda i: (0,))


def _dot(a, b):
    return jnp.dot(a, b, preferred_element_type=f32)


def _mlp_m_kernel(x1, x2, c0, c1, w1a, b1a, w2a, b2a, w1b, b1b, w2b, b2b,
                  W0, W1):
    n = x1.shape[0]
    bm = 2048
    grid = (n // bm,)

    def body(x1r, x2r, c0r, c1r, w1ar, b1ar, w2ar, b2ar, w1br, b1br, w2br,
             b2br, W0r, W1r, yb_o, ta0_o, ta1_o, tb0_o, tb1_o):
        ya = _dot(jax.nn.relu(_dot(x1r[...], w1ar[...]) + b1ar[...]), w2ar[...]) + b2ar[...]
        yb = _dot(jax.nn.relu(_dot(x2r[...], w1br[...]) + b1br[...]), w2br[...]) + b2br[...]
        d0 = (1.0 / jnp.sqrt(c0r[...] + 1.0))[:, None]
        d1 = (1.0 / jnp.sqrt(c1r[...] + 1.0))[:, None]
        yb_o[...] = yb
        ta0_o[...] = d0 * _dot(ya, W0r[...])
        ta1_o[...] = d1 * _dot(ya, W1r[...])
        tb0_o[...] = d0 * _dot(yb, W0r[...])
        tb1_o[...] = d1 * _dot(yb, W1r[...])

    sds = jax.ShapeDtypeStruct((n, F), f32)
    return pl.pallas_call(
        body,
        grid=grid,
        in_specs=[_row(bm), _row(bm), _vec(bm), _vec(bm),
                  _wmat(), _bvec(), _wmat(), _bvec(),
                  _wmat(), _bvec(), _wmat(), _bvec(), _wmat(), _wmat()],
        out_specs=[_row(bm)] * 5,
        out_shape=[sds] * 5,
    )(x1, x2, c0, c1, w1a, b1a, w2a, b2a, w1b, b1b, w2b, b2b, W0, W1)


def _mlp_d_kernel(x1, c0, c1, w1a, b1a, w2a, b2a, W0, W1):
    n = x1.shape[0]
    bm = 2048
    grid = (n // bm,)

    def body(x1r, c0r, c1r, w1ar, b1ar, w2ar, b2ar, W0r, W1r,
             y_o, t0_o, t1_o):
        y = _dot(jax.nn.relu(_dot(x1r[...], w1ar[...]) + b1ar[...]), w2ar[...]) + b2ar[...]
        d0 = (1.0 / jnp.sqrt(c0r[...] + 1.0))[:, None]
        d1 = (1.0 / jnp.sqrt(c1r[...] + 1.0))[:, None]
        y_o[...] = y
        t0_o[...] = d0 * _dot(y, W0r[...])
        t1_o[...] = d1 * _dot(y, W1r[...])

    sds = jax.ShapeDtypeStruct((n, F), f32)
    return pl.pallas_call(
        body,
        grid=grid,
        in_specs=[_row(bm), _vec(bm), _vec(bm),
                  _wmat(), _bvec(), _wmat(), _bvec(), _wmat(), _wmat()],
        out_specs=[_row(bm)] * 3,
        out_shape=[sds] * 3,
    )(x1, c0, c1, w1a, b1a, w2a, b2a, W0, W1)


def _mid_kernel(agg_h0, agg_h1, agg_e0, agg_e1, agg_sage,
                t0h, t1h, t0e, t1e, xe, c0, c1, csage,
                b0, b1, Wl, bl, Wr, V0, V1):
    n = agg_h0.shape[0]
    bm = 2048
    grid = (n // bm,)

    def body(ah0, ah1, ae0, ae1, asg, t0hr, t1hr, t0er, t1er, xer,
             c0r, c1r, csr, b0r, b1r, Wlr, blr, Wrr, V0r, V1r,
             ye_o, th0_o, th1_o, te0_o, te1_o):
        d0 = (1.0 / jnp.sqrt(c0r[...] + 1.0))[:, None]
        d1 = (1.0 / jnp.sqrt(c1r[...] + 1.0))[:, None]
        zh = (d0 * ah0[...] + d0 * t0hr[...] + b0r[...]
              + d1 * ah1[...] + d1 * t1hr[...] + b1r[...])
        mean = asg[...] / jnp.maximum(csr[...], 1.0)[:, None]
        ze = (d0 * ae0[...] + d0 * t0er[...] + b0r[...]
              + d1 * ae1[...] + d1 * t1er[...] + b1r[...]
              + _dot(mean, Wlr[...]) + blr[...] + _dot(xer[...], Wrr[...]))
        yh = jax.nn.relu(zh)
        ye = jax.nn.relu(ze)
        ye_o[...] = ye
        th0_o[...] = d0 * _dot(yh, V0r[...])
        th1_o[...] = d1 * _dot(yh, V1r[...])
        te0_o[...] = d0 * _dot(ye, V0r[...])
        te1_o[...] = d1 * _dot(ye, V1r[...])

    sds = jax.ShapeDtypeStruct((n, F), f32)
    return pl.pallas_call(
        body,
        grid=grid,
        in_specs=[_row(bm)] * 10 + [_vec(bm)] * 3
        + [_bvec(), _bvec(), _wmat(), _bvec(), _wmat(), _wmat(), _wmat()],
        out_specs=[_row(bm)] * 5,
        out_shape=[sds] * 5,
    )(agg_h0, agg_h1, agg_e0, agg_e1, agg_sage, t0h, t1h, t0e, t1e, xe,
      c0, c1, csage, b0, b1, Wl, bl, Wr, V0, V1)


def _post_kernel(agg_h0, agg_h1, agg_e0, agg_e1, agg_sage,
                 t0h, t1h, t0e, t1e, xe, c0, c1, csage,
                 b0, b1, Wl, bl, Wr, fW1, fW2, fb):
    n = agg_h0.shape[0]
    bm = 2048
    grid = (n // bm,)

    def body(ah0, ah1, ae0, ae1, asg, t0hr, t1hr, t0er, t1er, xer,
             c0r, c1r, csr, b0r, b1r, Wlr, blr, Wrr, fW1r, fW2r, fbr, z_o):
        d0 = (1.0 / jnp.sqrt(c0r[...] + 1.0))[:, None]
        d1 = (1.0 / jnp.sqrt(c1r[...] + 1.0))[:, None]
        zs = (d0 * ah0[...] + d0 * t0hr[...] + b0r[...]
              + d1 * ah1[...] + d1 * t1hr[...] + b1r[...])
        mean = asg[...] / jnp.maximum(csr[...], 1.0)[:, None]
        ze = (d0 * ae0[...] + d0 * t0er[...] + b0r[...]
              + d1 * ae1[...] + d1 * t1er[...] + b1r[...]
              + _dot(mean, Wlr[...]) + blr[...] + _dot(xer[...], Wrr[...]))
        z_o[...] = jax.nn.relu(_dot(zs, fW1r[...]) + _dot(ze, fW2r[...]) + fbr[...])

    return pl.pallas_call(
        body,
        grid=grid,
        in_specs=[_row(bm)] * 10 + [_vec(bm)] * 3
        + [_bvec(), _bvec(), _wmat(), _bvec(), _wmat(),
           _wmat(), _wmat(), _bvec()],
        out_specs=[_row(bm)],
        out_shape=[jax.ShapeDtypeStruct((n, F), f32)],
    )(agg_h0, agg_h1, agg_e0, agg_e1, agg_sage, t0h, t1h, t0e, t1e, xe,
      c0, c1, csage, b0, b1, Wl, bl, Wr, fW1, fW2, fb)[0]


def _bilinear_kernel(zm, Wbil, zd):
    bm = 2000
    grid = (NM // bm,)

    def body(zmr, Wr, zdr, o):
        t = _dot(zmr[...], Wr[...])
        o[...] = lax.dot_general(t, zdr[...], (((1,), (1,)), ((), ())),
                                 preferred_element_type=f32)

    return pl.pallas_call(
        body,
        grid=grid,
        in_specs=[_row(bm), _wmat(), pl.BlockSpec((ND, F), lambda i: (0, 0))],
        out_specs=[pl.BlockSpec((bm, ND), lambda i: (i, 0))],
        out_shape=[jax.ShapeDtypeStruct((NM, ND), f32)],
    )(zm, Wbil, zd)[0]


def kernel(m_x_sim, m_x_sem, d_x_sim, ei_mm, ei_mmr, ei_dd, ei_ddr, ei_md,
           ei_dm, params):
    p = params
    mxs = _pad_rows(m_x_sim, NMP)
    mxe = _pad_rows(m_x_sem, NMP)
    dxs = _pad_rows(d_x_sim, NDP)

    s_mm, d_mm = _pad_edges(ei_mm, EP_MM, NM)
    s_mmr, d_mmr = _pad_edges(ei_mmr, EP_MM, NM)
    s_dd, d_dd = _pad_edges(ei_dd, EP_DD, ND)
    s_ddr, d_ddr = _pad_edges(ei_ddr, EP_DD, ND)
    s_md, d_md = _pad_edges(ei_md, EP_MD, ND)
    s_dm, d_dm = _pad_edges(ei_dm, EP_MD, NM)

    ones_h = jnp.ones((CH,), f32)
    z1_h = jnp.zeros((NMP // NS,), f32)
    zrow_h = jnp.zeros((NMP // NS, F), f32)

    cnt_jobs = [
        dict(core=0, e=0, ndst=NMP, chunks=EP_MM // (NS * CH)),
        dict(core=0, e=2, ndst=NDP, chunks=EP_DD // (NS * CH)),
        dict(core=0, e=4, ndst=NDP, chunks=EP_MD // (NS * CH)),
        dict(core=1, e=1, ndst=NMP, chunks=EP_MM // (NS * CH)),
        dict(core=1, e=3, ndst=NDP, chunks=EP_DD // (NS * CH)),
        dict(core=1, e=5, ndst=NMP, chunks=EP_MD // (NS * CH)),
    ]
    cmm, cdd, cmd, cmmr, cddr, cdm = _make_counts(
        cnt_jobs, 6, max(j["chunks"] for j in cnt_jobs))(
        d_mm, d_mmr, d_dd, d_ddr, d_md, d_dm, ones_h, z1_h)

    me, t1_mm_h, t1_mmr_h, t1_mm_e, t1_mmr_e = _mlp_m_kernel(
        mxs, mxe, cmm, cmmr,
        p["msim_W1"], p["msim_b1"], p["msim_W2"], p["msim_b2"],
        p["msem_W1"], p["msem_b1"], p["msem_W2"], p["msem_b2"],
        p["g1_mm_W"], p["g1_mmr_W"])
    ds, t1_dd, t1_ddr = _mlp_d_kernel(
        dxs, cdd, cddr,
        p["dsim_W1"], p["dsim_b1"], p["dsim_W2"], p["dsim_b2"],
        p["g1_dd_W"], p["g1_ddr_W"])

    ch_mm = EP_MM // (NS * CH)
    ch_dd = EP_DD // (NS * CH)
    ch_md = EP_MD // (NS * CH)
    l1_jobs = [
        dict(core=0, t=0, e=0, ndst=NMP, chunks=ch_mm),
        dict(core=0, t=1, e=1, ndst=NMP, chunks=ch_mm),
        dict(core=0, t=4, e=2, ndst=NDP, chunks=ch_dd),
        dict(core=0, t=6, e=4, ndst=NDP, chunks=ch_md),
        dict(core=1, t=2, e=0, ndst=NMP, chunks=ch_mm),
        dict(core=1, t=3, e=1, ndst=NMP, chunks=ch_mm),
        dict(core=1, t=5, e=3, ndst=NDP, chunks=ch_dd),
        dict(core=1, t=7, e=5, ndst=NMP, chunks=ch_md),
    ]
    (a1_mm_h, a1_mmr_h, a1_dd, a1_md,
     a1_mm_e, a1_mmr_e, a1_ddr, a1_dm) = _make_prop(l1_jobs, 8, 6, ch_mm)(
        t1_mm_h, t1_mmr_h, t1_mm_e, t1_mmr_e, t1_dd, t1_ddr, me, ds,
        s_mm, s_mmr, s_dd, s_ddr, s_md, s_dm,
        d_mm, d_mmr, d_dd, d_ddr, d_md, d_dm,
        zrow_h)

    m_e1, t2_mm_h, t2_mmr_h, t2_mm_e, t2_mmr_e = _mid_kernel(
        a1_mm_h, a1_mmr_h, a1_mm_e, a1_mmr_e, a1_dm,
        t1_mm_h, t1_mmr_h, t1_mm_e, t1_mmr_e, me,
        cmm, cmmr, cdm,
        p["g1_mm_b"], p["g1_mmr_b"],
        p["g1_dm_Wl"], p["g1_dm_bl"], p["g1_dm_Wr"],
        p["g2_mm_W"], p["g2_mmr_W"])
    d_e1, t2_dd_h, t2_ddr_h, t2_dd_e, t2_ddr_e = _mid_kernel(
        a1_dd, a1_ddr, a1_dd, a1_ddr, a1_md,
        t1_dd, t1_ddr, t1_dd, t1_ddr, ds,
        cdd, cddr, cmd,
        p["g1_dd_b"], p["g1_ddr_b"],
        p["g1_md_Wl"], p["g1_md_bl"], p["g1_md_Wr"],
        p["g2_dd_W"], p["g2_ddr_W"])

    l2_jobs = [
        dict(core=0, t=0, e=0, ndst=NMP, chunks=ch_mm),
        dict(core=0, t=1, e=1, ndst=NMP, chunks=ch_mm),
        dict(core=0, t=4, e=2, ndst=NDP, chunks=ch_dd),
        dict(core=0, t=5, e=3, ndst=NDP, chunks=ch_dd),
        dict(core=0, t=8, e=4, ndst=NDP, chunks=ch_md),
        dict(core=1, t=2, e=0, ndst=NMP, chunks=ch_mm),
        dict(core=1, t=3, e=1, ndst=NMP, chunks=ch_mm),
        dict(core=1, t=6, e=2, ndst=NDP, chunks=ch_dd),
        dict(core=1, t=7, e=3, ndst=NDP, chunks=ch_dd),
        dict(core=1, t=9, e=5, ndst=NMP, chunks=ch_md),
    ]
    (a2_mm_h, a2_mmr_h, a2_dd_h, a2_ddr_h, a2_md,
     a2_mm_e, a2_mmr_e, a2_dd_e, a2_ddr_e, a2_dm) = _make_prop(l2_jobs, 10, 6, ch_mm)(
        t2_mm_h, t2_mmr_h, t2_mm_e, t2_mmr_e, t2_dd_h, t2_ddr_h,
        t2_dd_e, t2_ddr_e, m_e1, d_e1,
        s_mm, s_mmr, s_dd, s_ddr, s_md, s_dm,
        d_mm, d_mmr, d_dd, d_ddr, d_md, d_dm,
        zrow_h)

    fm_W1, fm_W2 = p["fm_W"][:F], p["fm_W"][F:]
    fd_W1, fd_W2 = p["fd_W"][:F], p["fd_W"][F:]
    zd = _post_kernel(
        a2_dd_h, a2_ddr_h, a2_dd_e, a2_ddr_e, a2_md,
        t2_dd_h, t2_ddr_h, t2_dd_e, t2_ddr_e, d_e1,
        cdd, cddr, cmd,
        p["g2_dd_b"], p["g2_ddr_b"],
        p["g2_md_Wl"], p["g2_md_bl"], p["g2_md_Wr"],
        fd_W1, fd_W2, p["fd_b"])
    zm = _post_kernel(
        a2_mm_h, a2_mmr_h, a2_mm_e, a2_mmr_e, a2_dm,
        t2_mm_h, t2_mmr_h, t2_mm_e, t2_mmr_e, m_e1,
        cmm, cmmr, cdm,
        p["g2_mm_b"], p["g2_mmr_b"],
        p["g2_dm_Wl"], p["g2_dm_bl"], p["g2_dm_Wr"],
        fm_W1, fm_W2, p["fm_b"])
    return _bilinear_kernel(zm, p["Wbil"], zd)
